# baseline (speedup 1.0000x reference)
import functools

import jax
import jax.numpy as jnp
from jax import lax
from jax.experimental import pallas as pl
from jax.experimental.pallas import tpu as pltpu
from jax.experimental.pallas import tpu_sc as plsc

DIM_ = 32
K_ = 2
MARGIN_ = 0.5
NW_ = 32
LANES_ = 16


def _make_cml(B):
    bpw = B // NW_
    mesh = plsc.VectorSubcoreMesh(core_axis_name="c", subcore_axis_name="s")

    @functools.partial(
        pl.kernel,
        mesh=mesh,
        out_type=jax.ShapeDtypeStruct((NW_, LANES_), jnp.float32),
        compiler_params=pltpu.CompilerParams(
            needs_layout_passes=False, use_tc_tiling_on_sc=False),
        scratch_types=[
            pltpu.VMEM((bpw,), jnp.int32),
            pltpu.VMEM((bpw,), jnp.int32),
            pltpu.VMEM((bpw,), jnp.int32),
            pltpu.VMEM((bpw, K_ * DIM_), jnp.float32),
            pltpu.VMEM((bpw, DIM_), jnp.float32),
            pltpu.VMEM((bpw, DIM_), jnp.float32),
            pltpu.VMEM((LANES_,), jnp.float32),
            pltpu.SemaphoreType.DMA,
            pltpu.SemaphoreType.DMA,
            pltpu.SemaphoreType.DMA,
        ],
    )
    def cml(uid_hbm, pid_hbm, nid_hbm, ut_hbm, it_hbm, out_hbm,
            uid_v, pid_v, nid_v, u_v, p_v, n_v, out_v, s0, s1, s2):
        wid = lax.axis_index("s") * 2 + lax.axis_index("c")
        base = wid * bpw
        pltpu.sync_copy(uid_hbm.at[pl.ds(base, bpw)], uid_v)
        pltpu.sync_copy(pid_hbm.at[pl.ds(base, bpw)], pid_v)
        pltpu.sync_copy(nid_hbm.at[pl.ds(base, bpw)], nid_v)
        cu = pltpu.async_copy(ut_hbm.at[uid_v], u_v, s0)
        cp = pltpu.async_copy(it_hbm.at[pid_v], p_v, s1)
        cn = pltpu.async_copy(it_hbm.at[nid_v], n_v, s2)
        cu.wait()
        cp.wait()
        cn.wait()

        def row(b, tot):
            u0a = u_v[b, pl.ds(0, 16)]
            u0b = u_v[b, pl.ds(16, 16)]
            u1a = u_v[b, pl.ds(32, 16)]
            u1b = u_v[b, pl.ds(48, 16)]
            pa = p_v[b, pl.ds(0, 16)]
            pb = p_v[b, pl.ds(16, 16)]
            na = n_v[b, pl.ds(0, 16)]
            nb = n_v[b, pl.ds(16, 16)]
            d0a = u0a - pa
            d0b = u0b - pb
            d1a = u1a - pa
            d1b = u1b - pb
            e0a = u0a - na
            e0b = u0b - nb
            e1a = u1a - na
            e1b = u1b - nb
            ep0 = d0a * d0a + d0b * d0b
            ep1 = d1a * d1a + d1b * d1b
            en0 = e0a * e0a + e0b * e0b
            en1 = e1a * e1a + e1b * e1b
            sp0 = jnp.sum(ep0)
            sp1 = jnp.sum(ep1)
            sn0 = jnp.sum(en0)
            sn1 = jnp.sum(en1)
            pos_d = jnp.minimum(sp0, sp1)
            neg_d = jnp.minimum(sn0, sn1)
            h = jnp.maximum(pos_d - neg_d + MARGIN_, 0.0)
            return tot + h

        total = lax.fori_loop(0, bpw, row, jnp.float32(0.0))
        scale = jnp.float32(1.0 / (LANES_ * B))
        out_v[...] = jnp.full((LANES_,), total * scale, dtype=jnp.float32)
        pltpu.sync_copy(out_v, out_hbm.at[wid])

    return cml


def kernel(user_ids, pos_ids, neg_ids, user_table, item_table):
    B = user_ids.shape[0]
    cml = _make_cml(B)
    partials = cml(user_ids.astype(jnp.int32), pos_ids.astype(jnp.int32),
                   neg_ids.astype(jnp.int32), user_table, item_table)
    return jnp.sum(partials)


# speedup vs baseline: 1.5289x; 1.5289x over previous
import functools

import jax
import jax.numpy as jnp
from jax import lax
from jax.experimental import pallas as pl
from jax.experimental.pallas import tpu as pltpu
from jax.experimental.pallas import tpu_sc as plsc

DIM_ = 32
K_ = 2
MARGIN_ = 0.5
NW_ = 32
LANES_ = 16
GROUP_ = 16
NBUF_ = 4
AHEAD_ = 3


def _make_cml(B):
    bpw = B // NW_
    ngroups = bpw // GROUP_
    mesh = plsc.VectorSubcoreMesh(core_axis_name="c", subcore_axis_name="s")

    @functools.partial(
        pl.kernel,
        mesh=mesh,
        out_type=jax.ShapeDtypeStruct((NW_, LANES_), jnp.float32),
        compiler_params=pltpu.CompilerParams(needs_layout_passes=False),
        scratch_types=[
            pltpu.VMEM((bpw,), jnp.int32),
            pltpu.VMEM((bpw,), jnp.int32),
            pltpu.VMEM((bpw,), jnp.int32),
            pltpu.VMEM((NBUF_ * GROUP_, K_ * DIM_), jnp.float32),
            pltpu.VMEM((NBUF_ * GROUP_, DIM_), jnp.float32),
            pltpu.VMEM((NBUF_ * GROUP_, DIM_), jnp.float32),
            pltpu.VMEM((LANES_,), jnp.float32),
            pltpu.SemaphoreType.DMA((NBUF_,)),
            pltpu.SemaphoreType.DMA((NBUF_,)),
            pltpu.SemaphoreType.DMA((NBUF_,)),
        ],
    )
    def cml(uid_hbm, pid_hbm, nid_hbm, ut_hbm, it_hbm, out_hbm,
            uid_v, pid_v, nid_v, u_v, p_v, n_v, out_v, su, sp, sn):
        wid = lax.axis_index("s") * 2 + lax.axis_index("c")
        base = wid * bpw
        pltpu.sync_copy(uid_hbm.at[pl.ds(base, bpw)], uid_v)
        pltpu.sync_copy(pid_hbm.at[pl.ds(base, bpw)], pid_v)
        pltpu.sync_copy(nid_hbm.at[pl.ds(base, bpw)], nid_v)

        def fire(g):
            buf = lax.rem(g, NBUF_)
            row0 = buf * GROUP_
            ug = uid_v[pl.ds(g * GROUP_, GROUP_)]
            pg = pid_v[pl.ds(g * GROUP_, GROUP_)]
            ng = nid_v[pl.ds(g * GROUP_, GROUP_)]
            for j in range(GROUP_):
                pltpu.async_copy(
                    ut_hbm.at[pl.ds(ug[j], 1)], u_v.at[pl.ds(row0 + j, 1)],
                    su.at[buf])
                pltpu.async_copy(
                    it_hbm.at[pl.ds(pg[j], 1)], p_v.at[pl.ds(row0 + j, 1)],
                    sp.at[buf])
                pltpu.async_copy(
                    it_hbm.at[pl.ds(ng[j], 1)], n_v.at[pl.ds(row0 + j, 1)],
                    sn.at[buf])

        def drain(g):
            buf = lax.rem(g, NBUF_)
            row0 = buf * GROUP_
            pltpu.make_async_copy(
                ut_hbm.at[pl.ds(0, GROUP_)], u_v.at[pl.ds(row0, GROUP_)],
                su.at[buf]).wait()
            pltpu.make_async_copy(
                it_hbm.at[pl.ds(0, GROUP_)], p_v.at[pl.ds(row0, GROUP_)],
                sp.at[buf]).wait()
            pltpu.make_async_copy(
                it_hbm.at[pl.ds(0, GROUP_)], n_v.at[pl.ds(row0, GROUP_)],
                sn.at[buf]).wait()

        def compute(g, tot):
            buf = lax.rem(g, NBUF_)
            row0 = buf * GROUP_
            for j in range(GROUP_):
                r = row0 + j
                u0a = u_v[r, pl.ds(0, 16)]
                u0b = u_v[r, pl.ds(16, 16)]
                u1a = u_v[r, pl.ds(32, 16)]
                u1b = u_v[r, pl.ds(48, 16)]
                pa = p_v[r, pl.ds(0, 16)]
                pb = p_v[r, pl.ds(16, 16)]
                na = n_v[r, pl.ds(0, 16)]
                nb = n_v[r, pl.ds(16, 16)]
                d0a = u0a - pa
                d0b = u0b - pb
                d1a = u1a - pa
                d1b = u1b - pb
                e0a = u0a - na
                e0b = u0b - nb
                e1a = u1a - na
                e1b = u1b - nb
                ep0 = d0a * d0a + d0b * d0b
                ep1 = d1a * d1a + d1b * d1b
                en0 = e0a * e0a + e0b * e0b
                en1 = e1a * e1a + e1b * e1b
                sp0 = jnp.sum(ep0)
                sp1 = jnp.sum(ep1)
                sn0 = jnp.sum(en0)
                sn1 = jnp.sum(en1)
                pos_d = jnp.minimum(sp0, sp1)
                neg_d = jnp.minimum(sn0, sn1)
                tot = tot + jnp.maximum(pos_d - neg_d + MARGIN_, 0.0)
            return tot

        for g in range(AHEAD_):
            fire(g)

        def body(g, tot):
            @pl.when(g + AHEAD_ < ngroups)
            def _():
                fire(g + AHEAD_)
            drain(g)
            return compute(g, tot)

        total = lax.fori_loop(0, ngroups, body, jnp.float32(0.0))
        scale = jnp.float32(1.0 / (LANES_ * B))
        out_v[...] = jnp.full((LANES_,), total * scale, dtype=jnp.float32)
        pltpu.sync_copy(out_v, out_hbm.at[wid])

    return cml


def kernel(user_ids, pos_ids, neg_ids, user_table, item_table):
    B = user_ids.shape[0]
    cml = _make_cml(B)
    partials = cml(user_ids.astype(jnp.int32), pos_ids.astype(jnp.int32),
                   neg_ids.astype(jnp.int32), user_table, item_table)
    return jnp.sum(partials)
